# baseline (speedup 1.0000x reference)
import functools

import jax
import jax.numpy as jnp
from jax import lax
from jax.experimental import pallas as pl
from jax.experimental.pallas import tpu as pltpu
from jax.experimental.pallas import tpu_sc as plsc

VOCAB = 1000000
D = 64
B = 1024
L = 200
EPS = 1e-5

NC = 2
NS = 16
NW = NC * NS
N = B * L
RPW = N // NW
CH = 128
NCH = RPW // CH


def _ln_rows(in_v, b, out_v, pe_v, g, bt, r0):
    g0, g1, g2, g3 = g
    b0, b1, b2, b3 = bt

    @plsc.parallel_loop(0, CH, step=1, unroll=24)
    def _row(i):
        p = lax.rem(r0 + i, L)
        pb = pl.multiple_of(p * D, 64)
        x0 = in_v[b, i, pl.ds(0, 16)] + pe_v[pl.ds(pb, 16)]
        x1 = in_v[b, i, pl.ds(16, 16)] + pe_v[pl.ds(pb + 16, 16)]
        x2 = in_v[b, i, pl.ds(32, 16)] + pe_v[pl.ds(pb + 32, 16)]
        x3 = in_v[b, i, pl.ds(48, 16)] + pe_v[pl.ds(pb + 48, 16)]
        s = (x0 + x1) + (x2 + x3)
        q = (x0 * x0 + x1 * x1) + (x2 * x2 + x3 * x3)
        mean = jnp.sum(s) * (1.0 / D)
        ex2 = jnp.sum(q) * (1.0 / D)
        t = ex2 - mean * mean + EPS
        ti = lax.bitcast_convert_type(t, jnp.int32)
        y = lax.bitcast_convert_type(jnp.int32(0x5F3759DF) - (ti >> 1),
                                     jnp.float32)
        y = y * (1.5 - 0.5 * t * y * y)
        y = y * (1.5 - 0.5 * t * y * y)
        out_v[b, i, pl.ds(0, 16)] = (x0 - mean) * (y * g0) + b0
        out_v[b, i, pl.ds(16, 16)] = (x1 - mean) * (y * g1) + b1
        out_v[b, i, pl.ds(32, 16)] = (x2 - mean) * (y * g2) + b2
        out_v[b, i, pl.ds(48, 16)] = (x3 - mean) * (y * g3) + b3


def _pe_ln(rows_hbm, pe_hbm, gb_hbm, out_hbm,
           in_v, out_v, pe_v, gb_v,
           psem, gsem0, gsem1, ssem0, ssem1):
    wid = lax.axis_index("s") * NC + lax.axis_index("c")
    base = pl.multiple_of(wid * RPW, 256)

    pltpu.async_copy(pe_hbm, pe_v, psem)
    pltpu.async_copy(gb_hbm, gb_v, psem).wait()
    pltpu.make_async_copy(pe_hbm, pe_v, psem).wait()

    g = tuple(gb_v[pl.ds(16 * j, 16)] for j in range(4))
    bt = tuple(gb_v[pl.ds(D + 16 * j, 16)] for j in range(4))

    gsems = (gsem0, gsem1)
    ssems = (ssem0, ssem1)

    def start_fetch(k, b, sem):
        off = pl.multiple_of(base + k * CH, 128)
        pltpu.async_copy(rows_hbm.at[pl.ds(off, CH)], in_v.at[b], sem)

    def wait_fetch(b, sem):
        pltpu.make_async_copy(rows_hbm.at[pl.ds(0, CH)], in_v.at[b],
                              sem).wait()

    def start_scatter(k, b, sem):
        off = pl.multiple_of(base + k * CH, 128)
        pltpu.async_copy(out_v.at[b], out_hbm.at[pl.ds(off, CH)], sem)

    def wait_scatter(b, sem):
        pltpu.make_async_copy(out_v.at[b], out_hbm.at[pl.ds(0, CH)],
                              sem).wait()

    start_fetch(0, 0, gsems[0])

    def step(k, b):
        nb = 1 - b

        @pl.when(k + 1 < NCH)
        def _prefetch():
            @pl.when(k >= 1)
            def _drain():
                wait_scatter(nb, ssems[nb])
            start_fetch(k + 1, nb, gsems[nb])

        wait_fetch(b, gsems[b])
        r0 = lax.rem(k * CH, L)
        _ln_rows(in_v, b, out_v, pe_v, g, bt, r0)
        start_scatter(k, b, ssems[b])

    def pair(p, _):
        step(2 * p, 0)
        step(2 * p + 1, 1)
        return 0

    lax.fori_loop(0, NCH // 2, pair, 0)
    wait_scatter(0, ssems[0])
    wait_scatter(1, ssems[1])


@jax.jit
def _run(rows, pe_flat, gb):
    mesh = plsc.VectorSubcoreMesh(core_axis_name="c", subcore_axis_name="s")
    return pl.kernel(
        _pe_ln,
        out_type=jax.ShapeDtypeStruct((N, D), jnp.float32),
        mesh=mesh,
        scratch_types=[
            pltpu.VMEM((2, CH, D), jnp.float32),
            pltpu.VMEM((2, CH, D), jnp.float32),
            pltpu.VMEM((L * D,), jnp.float32),
            pltpu.VMEM((2 * D,), jnp.float32),
            pltpu.SemaphoreType.DMA,
            pltpu.SemaphoreType.DMA,
            pltpu.SemaphoreType.DMA,
            pltpu.SemaphoreType.DMA,
            pltpu.SemaphoreType.DMA,
        ],
        compiler_params=pltpu.CompilerParams(needs_layout_passes=False),
    )(rows, pe_flat, gb)


def kernel(token_ids, table, gamma, beta, pe):
    ids = token_ids.astype(jnp.int32).reshape(-1)
    dnums = lax.GatherDimensionNumbers(
        offset_dims=(1,), collapsed_slice_dims=(0,), start_index_map=(0,))
    rows = lax.gather(table, ids[:, None], dnums, slice_sizes=(1, D),
                      mode=lax.GatherScatterMode.PROMISE_IN_BOUNDS)
    pe_flat = pe[0, :L, :].astype(jnp.float32).reshape(-1)
    gb = jnp.concatenate([gamma, beta]).astype(jnp.float32)
    out = _run(rows, pe_flat, gb)
    return out.reshape(B, L, D)


# speedup vs baseline: 1.0805x; 1.0805x over previous
import functools

import jax
import jax.numpy as jnp
from jax import lax
from jax.experimental import pallas as pl
from jax.experimental.pallas import tpu as pltpu
from jax.experimental.pallas import tpu_sc as plsc

VOCAB = 1000000
D = 64
B = 1024
L = 200
EPS = 1e-5

NC = 2
NS = 16
NW = NC * NS
N = B * L
RPW = N // NW
CH = 128
NCH = RPW // CH


def _ln_rows(in_v, b, out_v, pe_v, g, bt, r0):
    g0, g1, g2, g3 = g
    b0, b1, b2, b3 = bt

    @plsc.parallel_loop(0, CH, step=1, unroll=16)
    def _row(i):
        p = lax.rem(r0 + i, L)
        pb = pl.multiple_of(p * D, 64)
        x0 = in_v[b, i, pl.ds(0, 16)] + pe_v[pl.ds(pb, 16)]
        x1 = in_v[b, i, pl.ds(16, 16)] + pe_v[pl.ds(pb + 16, 16)]
        x2 = in_v[b, i, pl.ds(32, 16)] + pe_v[pl.ds(pb + 32, 16)]
        x3 = in_v[b, i, pl.ds(48, 16)] + pe_v[pl.ds(pb + 48, 16)]
        s = (x0 + x1) + (x2 + x3)
        q = (x0 * x0 + x1 * x1) + (x2 * x2 + x3 * x3)
        mean = jnp.sum(s) * (1.0 / D)
        ex2 = jnp.sum(q) * (1.0 / D)
        t = ex2 - mean * mean + EPS
        ti = lax.bitcast_convert_type(t, jnp.int32)
        y = lax.bitcast_convert_type(jnp.int32(0x5F3759DF) - (ti >> 1),
                                     jnp.float32)
        y = y * (1.5 - 0.5 * t * y * y)
        y = y * (1.5 - 0.5 * t * y * y)
        out_v[b, i, pl.ds(0, 16)] = (x0 - mean) * (y * g0) + b0
        out_v[b, i, pl.ds(16, 16)] = (x1 - mean) * (y * g1) + b1
        out_v[b, i, pl.ds(32, 16)] = (x2 - mean) * (y * g2) + b2
        out_v[b, i, pl.ds(48, 16)] = (x3 - mean) * (y * g3) + b3


def _pe_ln(rows_hbm, pe_hbm, gb_hbm, out_hbm,
           in_v, out_v, pe_v, gb_v,
           psem, gsem0, gsem1, ssem0, ssem1):
    wid = lax.axis_index("s") * NC + lax.axis_index("c")
    base = pl.multiple_of(wid * RPW, 256)

    pltpu.async_copy(pe_hbm, pe_v, psem)
    pltpu.async_copy(gb_hbm, gb_v, psem).wait()
    pltpu.make_async_copy(pe_hbm, pe_v, psem).wait()

    g = tuple(gb_v[pl.ds(16 * j, 16)] for j in range(4))
    bt = tuple(gb_v[pl.ds(D + 16 * j, 16)] for j in range(4))

    gsems = (gsem0, gsem1)
    ssems = (ssem0, ssem1)

    def start_fetch(k, b, sem):
        off = pl.multiple_of(base + k * CH, 128)
        pltpu.async_copy(rows_hbm.at[pl.ds(off, CH)], in_v.at[b], sem)

    def wait_fetch(b, sem):
        pltpu.make_async_copy(rows_hbm.at[pl.ds(0, CH)], in_v.at[b],
                              sem).wait()

    def start_scatter(k, b, sem):
        off = pl.multiple_of(base + k * CH, 128)
        pltpu.async_copy(out_v.at[b], out_hbm.at[pl.ds(off, CH)], sem)

    def wait_scatter(b, sem):
        pltpu.make_async_copy(out_v.at[b], out_hbm.at[pl.ds(0, CH)],
                              sem).wait()

    start_fetch(0, 0, gsems[0])

    def step(k, b):
        nb = 1 - b

        @pl.when(k + 1 < NCH)
        def _prefetch():
            @pl.when(k >= 1)
            def _drain():
                wait_scatter(nb, ssems[nb])
            start_fetch(k + 1, nb, gsems[nb])

        wait_fetch(b, gsems[b])
        r0 = lax.rem(k * CH, L)
        _ln_rows(in_v, b, out_v, pe_v, g, bt, r0)
        start_scatter(k, b, ssems[b])

    def pair(p, _):
        step(2 * p, 0)
        step(2 * p + 1, 1)
        return 0

    lax.fori_loop(0, NCH // 2, pair, 0)
    wait_scatter(0, ssems[0])
    wait_scatter(1, ssems[1])


@jax.jit
def _run(rows, pe_flat, gb):
    mesh = plsc.VectorSubcoreMesh(core_axis_name="c", subcore_axis_name="s")
    return pl.kernel(
        _pe_ln,
        out_type=jax.ShapeDtypeStruct((N, D), jnp.float32),
        mesh=mesh,
        scratch_types=[
            pltpu.VMEM((2, CH, D), jnp.float32),
            pltpu.VMEM((2, CH, D), jnp.float32),
            pltpu.VMEM((L * D,), jnp.float32),
            pltpu.VMEM((2 * D,), jnp.float32),
            pltpu.SemaphoreType.DMA,
            pltpu.SemaphoreType.DMA,
            pltpu.SemaphoreType.DMA,
            pltpu.SemaphoreType.DMA,
            pltpu.SemaphoreType.DMA,
        ],
        compiler_params=pltpu.CompilerParams(needs_layout_passes=False),
    )(rows, pe_flat, gb)


def kernel(token_ids, table, gamma, beta, pe):
    ids = token_ids.astype(jnp.int32).reshape(-1)
    dnums = lax.GatherDimensionNumbers(
        offset_dims=(1,), collapsed_slice_dims=(0,), start_index_map=(0,))
    rows = lax.gather(table, ids[:, None], dnums, slice_sizes=(1, D),
                      mode=lax.GatherScatterMode.PROMISE_IN_BOUNDS)
    pe_flat = pe[0, :L, :].astype(jnp.float32).reshape(-1)
    gb = jnp.concatenate([gamma, beta]).astype(jnp.float32)
    out = _run(rows, pe_flat, gb)
    return out.reshape(B, L, D)
